# Initial kernel scaffold; baseline (speedup 1.0000x reference)
import functools

import jax
import jax.numpy as jnp
from jax import lax
from jax.experimental import pallas as pl
from jax.experimental.pallas import tpu as pltpu
from jax.experimental.pallas import tpu_sc as plsc

DIM = 32
B_ROWS = 16384
B_COLS = 26
B_TOTAL = B_ROWS * B_COLS
NW = 32
PER_W = B_TOTAL // NW
GRP = 128
GRPS_PER_CHUNK = 8
CHUNK = GRP * GRPS_PER_CHUNK
NCHUNK = PER_W // CHUNK

_mesh = plsc.VectorSubcoreMesh(core_axis_name="c", subcore_axis_name="s")


@functools.partial(
    pl.kernel,
    mesh=_mesh,
    out_type=jax.ShapeDtypeStruct((B_TOTAL, DIM), jnp.float32),
    scratch_types=[
        pltpu.VMEM((GRPS_PER_CHUNK, GRP), jnp.int32),
        pltpu.VMEM((CHUNK, DIM), jnp.float32),
        pltpu.SemaphoreType.DMA,
    ],
)
def _gather_kernel(idx_hbm, table_hbm, out_hbm, idx_v, rows_v, sem):
    wid = lax.axis_index("s") * 2 + lax.axis_index("c")
    chunk0 = wid * NCHUNK

    def body(c, carry):
        blk = chunk0 + c
        pltpu.sync_copy(idx_hbm.at[blk], idx_v)
        handles = []
        for j in range(GRPS_PER_CHUNK):
            handles.append(
                pltpu.async_copy(
                    table_hbm.at[idx_v.at[j]],
                    rows_v.at[pl.ds(j * GRP, GRP)],
                    sem,
                )
            )
        for h in handles:
            h.wait()
        off = pl.multiple_of(blk * CHUNK, CHUNK)
        pltpu.sync_copy(rows_v, out_hbm.at[pl.ds(off, CHUNK)])
        return carry

    lax.fori_loop(0, NCHUNK, body, 0)


def kernel(inputs, table):
    flat_idx = inputs.astype(jnp.int32).reshape(NW * NCHUNK, GRPS_PER_CHUNK, GRP)
    out = _gather_kernel(flat_idx, table)
    return out.reshape(B_ROWS, B_COLS, DIM)


# speedup vs baseline: 1.5473x; 1.5473x over previous
import functools

import jax
import jax.numpy as jnp
from jax import lax
from jax.experimental import pallas as pl
from jax.experimental.pallas import tpu as pltpu
from jax.experimental.pallas import tpu_sc as plsc

DIM = 32
B_ROWS = 16384
B_COLS = 26
B_TOTAL = B_ROWS * B_COLS
NW = 32
PER_W = B_TOTAL // NW
GRP = 128
GRPS_PER_CHUNK = 8
CHUNK = GRP * GRPS_PER_CHUNK
NCHUNK = PER_W // CHUNK

_mesh = plsc.VectorSubcoreMesh(core_axis_name="c", subcore_axis_name="s")


@functools.partial(
    pl.kernel,
    mesh=_mesh,
    compiler_params=pltpu.CompilerParams(use_tc_tiling_on_sc=False),
    out_type=jax.ShapeDtypeStruct((B_TOTAL, DIM), jnp.float32),
    scratch_types=[
        pltpu.VMEM((GRPS_PER_CHUNK, GRP), jnp.int32),
        pltpu.VMEM((CHUNK, DIM), jnp.float32),
        pltpu.SemaphoreType.DMA,
    ],
)
def _gather_kernel(idx_hbm, table_hbm, out_hbm, idx_v, rows_v, sem):
    wid = lax.axis_index("s") * 2 + lax.axis_index("c")
    chunk0 = wid * NCHUNK

    def body(c, carry):
        blk = chunk0 + c
        pltpu.sync_copy(idx_hbm.at[blk], idx_v)
        handles = []
        for j in range(GRPS_PER_CHUNK):
            handles.append(
                pltpu.async_copy(
                    table_hbm.at[idx_v.at[j]],
                    rows_v.at[pl.ds(j * GRP, GRP)],
                    sem,
                )
            )
        for h in handles:
            h.wait()
        off = pl.multiple_of(blk * CHUNK, CHUNK)
        pltpu.sync_copy(rows_v, out_hbm.at[pl.ds(off, CHUNK)])
        return carry

    lax.fori_loop(0, NCHUNK, body, 0)


def kernel(inputs, table):
    flat_idx = inputs.astype(jnp.int32).reshape(NW * NCHUNK, GRPS_PER_CHUNK, GRP)
    out = _gather_kernel(flat_idx, table)
    return out.reshape(B_ROWS, B_COLS, DIM)


# speedup vs baseline: 1.5698x; 1.0145x over previous
import functools

import jax
import jax.numpy as jnp
from jax import lax
from jax.experimental import pallas as pl
from jax.experimental.pallas import tpu as pltpu
from jax.experimental.pallas import tpu_sc as plsc

DIM = 32
B_ROWS = 16384
B_COLS = 26
B_TOTAL = B_ROWS * B_COLS
NW = 32
PER_W = B_TOTAL // NW
GRP = 128
GRPS_PER_W = PER_W // GRP
GRPS_PER_SLOT = 2
SLOT = GRP * GRPS_PER_SLOT
NBUF = 4
NC = PER_W // SLOT
NT = NC // NBUF

_mesh = plsc.VectorSubcoreMesh(core_axis_name="c", subcore_axis_name="s")


@functools.partial(
    pl.kernel,
    mesh=_mesh,
    compiler_params=pltpu.CompilerParams(use_tc_tiling_on_sc=False),
    out_type=jax.ShapeDtypeStruct((B_TOTAL, DIM), jnp.float32),
    scratch_types=(
        [pltpu.VMEM((GRPS_PER_W, GRP), jnp.int32)]
        + [pltpu.VMEM((SLOT, DIM), jnp.float32) for _ in range(NBUF)]
        + [pltpu.SemaphoreType.DMA for _ in range(2 * NBUF)]
    ),
)
def _gather_kernel(idx_hbm, table_hbm, out_hbm, idx_v, *bufs_and_sems):
    rows = bufs_and_sems[:NBUF]
    gsem = bufs_and_sems[NBUF:2 * NBUF]
    osem = bufs_and_sems[2 * NBUF:]

    wid = lax.axis_index("s") * 2 + lax.axis_index("c")
    grp0 = wid * GRPS_PER_W
    idx0 = wid * PER_W

    pltpu.sync_copy(idx_hbm.at[pl.ds(grp0, GRPS_PER_W)], idx_v)

    def start_gathers(c, s):
        for j in range(GRPS_PER_SLOT):
            pltpu.async_copy(
                table_hbm.at[idx_v.at[c * GRPS_PER_SLOT + j]],
                rows[s].at[pl.ds(j * GRP, GRP)],
                gsem[s],
            )

    def wait_gathers(s):
        for j in range(GRPS_PER_SLOT):
            pltpu.make_async_copy(
                table_hbm.at[idx_v.at[j]],
                rows[s].at[pl.ds(j * GRP, GRP)],
                gsem[s],
            ).wait()

    def start_store(c, s):
        off = pl.multiple_of(idx0 + c * SLOT, SLOT)
        pltpu.async_copy(rows[s], out_hbm.at[pl.ds(off, SLOT)], osem[s])

    def wait_store(c, s):
        off = pl.multiple_of(idx0 + c * SLOT, SLOT)
        pltpu.make_async_copy(
            rows[s], out_hbm.at[pl.ds(off, SLOT)], osem[s]
        ).wait()

    def body(t, carry):
        for s in range(NBUF):
            c = t * NBUF + s
            if s == 0:
                @pl.when(t > 0)
                def _():
                    wait_store(c - NBUF, s)
            else:
                @pl.when(t > 0)
                def _():
                    wait_store(c - NBUF, s)
            start_gathers(c, s)
            sp = (s - 1) % NBUF
            if s == 0:
                @pl.when(t > 0)
                def _():
                    wait_gathers(sp)
                    start_store(c - 1, sp)
            else:
                wait_gathers(sp)
                start_store(c - 1, sp)
        return carry

    lax.fori_loop(0, NT, body, 0)

    wait_gathers(NBUF - 1)
    start_store(NC - 1, NBUF - 1)
    for s in range(NBUF):
        wait_store(NC - NBUF + s, s)


def kernel(inputs, table):
    flat_idx = inputs.astype(jnp.int32).reshape(NW * GRPS_PER_W, GRP)
    out = _gather_kernel(flat_idx, table)
    return out.reshape(B_ROWS, B_COLS, DIM)
